# Initial kernel scaffold; baseline (speedup 1.0000x reference)
import jax
import jax.numpy as jnp
from jax.experimental import pallas as pl


def kernel(node_embs, edge_index, edge_embs, edge_weights, W_node, W_e1, b_e1, W_e2, b_e2, W_n1, b_n1, W_n2, b_n2):
    raise NotImplementedError("write your pallas kernel here")


# speedup vs baseline: 2.0820x; 2.0820x over previous
import functools
import math

import jax
import jax.numpy as jnp
from jax import lax
from jax.experimental import pallas as pl
from jax.experimental.pallas import tpu as pltpu
from jax.experimental.pallas import tpu_sc as plsc

N = 10000
E = 320000
H = 128
NF = 128
G = 16
CUTOFF = 0.5
SHIFT = math.log(2.0)

_info = plsc.get_sparse_core_info()
NC = _info.num_cores
NS = _info.num_subcores
NW = NC * NS
CH = 128
NCH = E // CH
TRIPS = (NCH + NW - 1) // NW
ZR = 624


def _ssp(x):
    return jnp.maximum(x, 0.0) + jnp.log1p(jnp.exp(-jnp.abs(x))) - SHIFT


def _h_body(x_ref, w_ref, o_ref):
    o_ref[...] = jnp.dot(x_ref[...], w_ref[...],
                         preferred_element_type=jnp.float32)


def _compute_h(node_embs, W_node):
    blk = 1000
    return pl.pallas_call(
        _h_body,
        grid=(N // blk,),
        in_specs=[
            pl.BlockSpec((blk, H), lambda i: (i, 0)),
            pl.BlockSpec((H, NF), lambda i: (0, 0)),
        ],
        out_specs=pl.BlockSpec((blk, NF), lambda i: (i, 0)),
        out_shape=jax.ShapeDtypeStruct((N, NF), jnp.float32),
    )(node_embs, W_node)


_EB = 2560
_EWR = _EB // 128


def _edge_body(eb_ref, we1_ref, be1_ref, we2_ref, be2_ref, w_ref):
    x1 = jnp.dot(eb_ref[...], we1_ref[...],
                 preferred_element_type=jnp.float32) + be1_ref[...]
    x2 = jnp.dot(_ssp(x1), we2_ref[...],
                 preferred_element_type=jnp.float32) + be2_ref[...]
    w_ref[...] = x2


def _compute_edge_mlp(edge_embs, W_e1, b_e1, W_e2, b_e2):
    return pl.pallas_call(
        _edge_body,
        grid=(E // _EB,),
        in_specs=[
            pl.BlockSpec((_EB, G), lambda i: (i, 0)),
            pl.BlockSpec((G, NF), lambda i: (0, 0)),
            pl.BlockSpec((1, NF), lambda i: (0, 0)),
            pl.BlockSpec((NF, NF), lambda i: (0, 0)),
            pl.BlockSpec((1, NF), lambda i: (0, 0)),
        ],
        out_specs=pl.BlockSpec((_EB, NF), lambda i: (i, 0)),
        out_shape=jax.ShapeDtypeStruct((E, NF), jnp.float32),
    )(edge_embs, W_e1, b_e1, W_e2, b_e2)


def _cut_body(ew_ref, cut_ref):
    ew = ew_ref[...]
    cut = 0.5 * (jnp.cos(ew * (math.pi / CUTOFF)) + 1.0)
    cut_ref[...] = cut * (ew < CUTOFF).astype(jnp.float32)


def _compute_cutoffs(ew2d):
    return pl.pallas_call(
        _cut_body,
        out_shape=jax.ShapeDtypeStruct((E // 128, 128), jnp.float32),
    )(ew2d)


def _make_sc_kernel():
    mesh = plsc.VectorSubcoreMesh(core_axis_name="c", subcore_axis_name="s")

    @functools.partial(
        pl.kernel,
        mesh=mesh,
        out_type=[
            jax.ShapeDtypeStruct((E, NF), jnp.float32),
            jax.ShapeDtypeStruct((2 * N, NF), jnp.float32),
        ],
        scratch_types=[
            pltpu.VMEM((CH,), jnp.int32),
            pltpu.VMEM((CH,), jnp.int32),
            pltpu.VMEM((CH, NF), jnp.float32),
            pltpu.VMEM((CH, NF), jnp.float32),
            pltpu.VMEM((CH,), jnp.float32),
            pltpu.VMEM_SHARED((N, NF), jnp.float32),
            pltpu.SemaphoreType.DMA,
        ],
    )
    def sc_kernel(h_hbm, is_hbm, js_hbm, w_hbm, cut_hbm,
                  ne_hbm, part_hbm,
                  is_v, js_v, rows_v, w_v, cut_v, acc_sh, sem):
        cid = lax.axis_index("c")
        sid = lax.axis_index("s")
        wid = sid * NC + cid

        def _zb(i, carry):
            for k in range(NF // 16):
                rows_v[i, pl.ds(k * 16, 16)] = jnp.zeros((16,), jnp.float32)
            return carry
        lax.fori_loop(0, CH, _zb, 0)
        zbase = sid * ZR
        nfull = ZR // CH
        for r in range(nfull):
            pltpu.sync_copy(rows_v, acc_sh.at[pl.ds(zbase + r * CH, CH)])
        rem = ZR - nfull * CH
        if rem:
            pltpu.sync_copy(rows_v.at[pl.ds(0, rem)],
                            acc_sh.at[pl.ds(zbase + nfull * CH, rem)])

        @pl.when(sid == NS - 1)
        def _zero_tail():
            pltpu.sync_copy(rows_v.at[pl.ds(0, N - NS * ZR)],
                            acc_sh.at[pl.ds(NS * ZR, N - NS * ZR)])
        plsc.subcore_barrier()

        def _chunk(t, carry):
            c = wid + NW * t

            @pl.when(c < NCH)
            def _():
                base = c * CH
                pltpu.sync_copy(is_hbm.at[pl.ds(base, CH)], is_v)
                pltpu.sync_copy(js_hbm.at[pl.ds(base, CH)], js_v)
                pltpu.async_copy(h_hbm.at[is_v], rows_v, sem).wait()
                pltpu.sync_copy(w_hbm.at[pl.ds(base, CH)], w_v)
                pltpu.sync_copy(cut_hbm.at[c], cut_v)

                def _row16(j, rcarry):
                    cv = cut_v[pl.ds(j * 16, 16)]
                    for r in range(16):
                        cs = cv[r]
                        i = j * 16 + r
                        for k in range(NF // 16):
                            sl = pl.ds(k * 16, 16)
                            rows_v[i, sl] = rows_v[i, sl] * w_v[i, sl] * cs
                    return rcarry
                lax.fori_loop(0, CH // 16, _row16, 0)

                pltpu.sync_copy(rows_v, ne_hbm.at[pl.ds(base, CH)])
                pltpu.sync_copy(rows_v, acc_sh.at[js_v], add=True)
            return carry
        lax.fori_loop(0, TRIPS, _chunk, 0)

        plsc.subcore_barrier()
        pltpu.sync_copy(acc_sh.at[pl.ds(sid * ZR, ZR)],
                        part_hbm.at[pl.ds(cid * N + sid * ZR, ZR)])

        @pl.when(sid == NS - 1)
        def _flush_tail():
            pltpu.sync_copy(acc_sh.at[pl.ds(NS * ZR, N - NS * ZR)],
                            part_hbm.at[pl.ds(cid * N + NS * ZR, N - NS * ZR)])

    return sc_kernel


_sc_kernel = _make_sc_kernel()


def _node_body(pa_ref, pb_ref, ne_ref, w1_ref, b1_ref, w2_ref, b2_ref,
               o_ref):
    acc = pa_ref[...] + pb_ref[...]
    t = jnp.dot(_ssp(jnp.dot(acc, w1_ref[...],
                             preferred_element_type=jnp.float32)
                     + b1_ref[...]),
                w2_ref[...], preferred_element_type=jnp.float32) + b2_ref[...]
    o_ref[...] = ne_ref[...] + t


def _node_update(partial, node_embs, W_n1, b_n1, W_n2, b_n2):
    blk = 1000
    nblk = N // blk
    return pl.pallas_call(
        _node_body,
        grid=(nblk,),
        in_specs=[
            pl.BlockSpec((blk, NF), lambda i: (i, 0)),
            pl.BlockSpec((blk, NF), lambda i: (i + nblk, 0)),
            pl.BlockSpec((blk, H), lambda i: (i, 0)),
            pl.BlockSpec((NF, H), lambda i: (0, 0)),
            pl.BlockSpec((1, H), lambda i: (0, 0)),
            pl.BlockSpec((H, H), lambda i: (0, 0)),
            pl.BlockSpec((1, H), lambda i: (0, 0)),
        ],
        out_specs=pl.BlockSpec((blk, H), lambda i: (i, 0)),
        out_shape=jax.ShapeDtypeStruct((N, H), jnp.float32),
    )(partial, partial, node_embs, W_n1, b_n1, W_n2, b_n2)


def kernel(node_embs, edge_index, edge_embs, edge_weights,
           W_node, W_e1, b_e1, W_e2, b_e2, W_n1, b_n1, W_n2, b_n2):
    node_is = edge_index[0]
    node_js = edge_index[1]
    ew2d = edge_weights.reshape(E // 128, 128)

    h = _compute_h(node_embs, W_node)
    W_edges = _compute_edge_mlp(
        edge_embs, W_e1, b_e1.reshape(1, NF), W_e2, b_e2.reshape(1, NF))
    cut2d = _compute_cutoffs(ew2d)

    new_edge_embs, partial = _sc_kernel(h, node_is, node_js, W_edges, cut2d)

    new_node_embs = _node_update(partial, node_embs, W_n1,
                                 b_n1.reshape(1, H), W_n2, b_n2.reshape(1, H))
    return (new_node_embs, new_edge_embs)
